# baseline (speedup 1.0000x reference)
import functools

import jax
import jax.numpy as jnp
from jax.experimental import pallas as pl
from jax.experimental.pallas import tpu as pltpu


def _fused_route_kernel(x0_ref, h0_ref, wq_ref, wk_ref, slot_ref, out_ref,
                        *, scale):
    h0 = h0_ref[0]
    x0 = x0_ref[0]
    wq = wq_ref[...]
    wk = wk_ref[...]
    slot_logits = slot_ref[...]

    q = jax.lax.dot_general(h0, wq, (((1,), (1,)), ((), ())),
                            preferred_element_type=jnp.float32)
    k = jax.lax.dot_general(x0, wk, (((1,), (1,)), ((), ())),
                            preferred_element_type=jnp.float32)

    s = jax.lax.dot_general(q.astype(jnp.bfloat16), k.astype(jnp.bfloat16),
                            (((1,), (1,)), ((), ())),
                            preferred_element_type=jnp.float32) * scale

    row_max = jnp.max(s, axis=-1, keepdims=True)
    scores = jnp.log(jnp.sum(jnp.exp(s - row_max), axis=-1)) + row_max[:, 0]

    smax = jnp.max(scores)
    e = jnp.exp(scores - smax)
    w_keep = e / jnp.sum(e)

    sl_max = jnp.max(slot_logits, axis=-1, keepdims=True)
    se = jnp.exp(slot_logits - sl_max)
    slot_w = se / jnp.sum(se, axis=-1, keepdims=True)

    w = slot_w * w_keep[None, :]
    z = jnp.sum(w, axis=-1, keepdims=True) + 1e-6
    out = jax.lax.dot_general(w.astype(jnp.bfloat16), h0,
                              (((1,), (0,)), ((), ())),
                              preferred_element_type=jnp.float32)
    out_ref[0] = out / z


def kernel(X0_patches, H0_patches, Wq, Wk, slot_logits):
    B, N, D = X0_patches.shape
    QK = Wq.shape[0]
    M = slot_logits.shape[0]
    scale = QK ** (-0.5)

    x0_bf = X0_patches.astype(jnp.bfloat16)
    h0_bf = H0_patches.astype(jnp.bfloat16)
    wq_bf = Wq.astype(jnp.bfloat16)
    wk_bf = Wk.astype(jnp.bfloat16)

    return pl.pallas_call(
        functools.partial(_fused_route_kernel, scale=scale),
        grid=(B,),
        in_specs=[
            pl.BlockSpec((1, N, D), lambda b: (b, 0, 0)),
            pl.BlockSpec((1, N, D), lambda b: (b, 0, 0)),
            pl.BlockSpec((QK, D), lambda b: (0, 0)),
            pl.BlockSpec((QK, D), lambda b: (0, 0)),
            pl.BlockSpec((M, N), lambda b: (0, 0)),
        ],
        out_specs=pl.BlockSpec((1, M, D), lambda b: (b, 0, 0)),
        out_shape=jax.ShapeDtypeStruct((B, M, D), jnp.float32),
        compiler_params=pltpu.CompilerParams(
            dimension_semantics=("parallel",),
        ),
    )(x0_bf, h0_bf, wq_bf, wk_bf, slot_logits)


# speedup vs baseline: 1.7613x; 1.7613x over previous
import functools

import jax
import jax.numpy as jnp
from jax.experimental import pallas as pl
from jax.experimental.pallas import tpu as pltpu


def _fused_route_kernel(x0_ref, h0_ref, wq_ref, wk_ref, slot_ref, out_ref,
                        *, scale):
    h0 = h0_ref[0].astype(jnp.bfloat16)
    x0 = x0_ref[0].astype(jnp.bfloat16)
    wq = wq_ref[...].astype(jnp.bfloat16)
    wk = wk_ref[...].astype(jnp.bfloat16)
    slot_logits = slot_ref[...]

    q = jax.lax.dot_general(h0, wq, (((1,), (1,)), ((), ())),
                            preferred_element_type=jnp.float32)
    k = jax.lax.dot_general(x0, wk, (((1,), (1,)), ((), ())),
                            preferred_element_type=jnp.float32)

    s = jax.lax.dot_general(q.astype(jnp.bfloat16), k.astype(jnp.bfloat16),
                            (((1,), (1,)), ((), ())),
                            preferred_element_type=jnp.float32) * scale

    row_max = jnp.max(s, axis=-1, keepdims=True)
    scores = jnp.log(jnp.sum(jnp.exp(s - row_max), axis=-1)) + row_max[:, 0]

    smax = jnp.max(scores)
    e = jnp.exp(scores - smax)
    w_keep = e / jnp.sum(e)

    sl_max = jnp.max(slot_logits, axis=-1, keepdims=True)
    se = jnp.exp(slot_logits - sl_max)
    slot_w = se / jnp.sum(se, axis=-1, keepdims=True)

    w = slot_w * w_keep[None, :]
    z = jnp.sum(w, axis=-1, keepdims=True) + 1e-6
    out = jax.lax.dot_general(w.astype(jnp.bfloat16), h0,
                              (((1,), (0,)), ((), ())),
                              preferred_element_type=jnp.float32)
    out_ref[0] = out / z


def kernel(X0_patches, H0_patches, Wq, Wk, slot_logits):
    B, N, D = X0_patches.shape
    QK = Wq.shape[0]
    M = slot_logits.shape[0]
    scale = QK ** (-0.5)

    return pl.pallas_call(
        functools.partial(_fused_route_kernel, scale=scale),
        grid=(B,),
        in_specs=[
            pl.BlockSpec((1, N, D), lambda b: (b, 0, 0)),
            pl.BlockSpec((1, N, D), lambda b: (b, 0, 0)),
            pl.BlockSpec((QK, D), lambda b: (0, 0)),
            pl.BlockSpec((QK, D), lambda b: (0, 0)),
            pl.BlockSpec((M, N), lambda b: (0, 0)),
        ],
        out_specs=pl.BlockSpec((1, M, D), lambda b: (b, 0, 0)),
        out_shape=jax.ShapeDtypeStruct((B, M, D), jnp.float32),
        compiler_params=pltpu.CompilerParams(
            dimension_semantics=("parallel",),
        ),
    )(X0_patches, H0_patches, Wq, Wk, slot_logits)
